# baseline (device time: 18472 ns/iter reference)
import jax
import jax.numpy as jnp
from jax import lax
from jax.experimental import pallas as pl
from jax.experimental.pallas import tpu as pltpu


def kernel(x, pi):
    def body(pi_ref, x_ref, out_ref, send_sem, recv_sem, copy_sem):
        my_x = lax.axis_index("x")
        my_y = lax.axis_index("y")
        my_z = lax.axis_index("z")
        tgt = pi_ref[my_x]

        is_remote = tgt != my_x

        @pl.when(is_remote)
        def _():
            barrier = pltpu.get_barrier_semaphore()
            pl.semaphore_signal(
                barrier,
                inc=1,
                device_id=(tgt, my_y, my_z),
                device_id_type=pl.DeviceIdType.MESH,
            )
            pl.semaphore_wait(barrier, 1)
            rdma = pltpu.make_async_remote_copy(
                src_ref=x_ref,
                dst_ref=out_ref,
                send_sem=send_sem,
                recv_sem=recv_sem,
                device_id=(tgt, my_y, my_z),
                device_id_type=pl.DeviceIdType.MESH,
            )
            rdma.start()
            rdma.wait()

        @pl.when(jnp.logical_not(is_remote))
        def _():
            copy = pltpu.make_async_copy(x_ref, out_ref, copy_sem)
            copy.start()
            copy.wait()

    return pl.pallas_call(
        body,
        out_shape=jax.ShapeDtypeStruct(x.shape, x.dtype),
        in_specs=[
            pl.BlockSpec(memory_space=pltpu.SMEM),
            pl.BlockSpec(memory_space=pl.ANY),
        ],
        out_specs=pl.BlockSpec(memory_space=pltpu.HBM),
        scratch_shapes=[
            pltpu.SemaphoreType.DMA,
            pltpu.SemaphoreType.DMA,
            pltpu.SemaphoreType.DMA,
        ],
        compiler_params=pltpu.CompilerParams(collective_id=0),
    )(pi, x)


# device time: 16253 ns/iter; 1.1365x vs baseline; 1.1365x over previous
import jax
import jax.numpy as jnp
from jax import lax
from jax.experimental import pallas as pl
from jax.experimental.pallas import tpu as pltpu

_R = 128


def kernel(x, pi):
    def body(pi_ref, x_ref, out_ref,
             sa_send, sa_recv, sb_send, sb_recv,
             fy_send, fy_recv, fz_send, fz_recv, copy_sem):
        my_x = lax.axis_index("x")
        my_y = lax.axis_index("y")
        my_z = lax.axis_index("z")
        tgt = pi_ref[my_x]
        is_remote = tgt != my_x

        k = 2 * my_y + my_z
        kd = 3 - k

        partner = (tgt, my_y, my_z)
        y_nbr = (my_x, 1 - my_y, my_z)
        z_nbr = (my_x, my_y, 1 - my_z)

        @pl.when(is_remote)
        def _():
            barrier = pltpu.get_barrier_semaphore()
            for dev in (partner, y_nbr, z_nbr):
                pl.semaphore_signal(
                    barrier, inc=1,
                    device_id=dev, device_id_type=pl.DeviceIdType.MESH,
                )
            pl.semaphore_wait(barrier, 3)

            rdma_a = pltpu.make_async_remote_copy(
                src_ref=x_ref.at[:, pl.ds(k * _R, _R), :],
                dst_ref=out_ref.at[:, pl.ds(k * _R, _R), :],
                send_sem=sa_send, recv_sem=sa_recv,
                device_id=partner, device_id_type=pl.DeviceIdType.MESH,
            )
            rdma_a.start()
            rdma_b = pltpu.make_async_remote_copy(
                src_ref=x_ref.at[:, pl.ds(kd * _R, _R), :],
                dst_ref=out_ref.at[:, pl.ds(kd * _R, _R), :],
                send_sem=sb_send, recv_sem=sb_recv,
                device_id=partner, device_id_type=pl.DeviceIdType.MESH,
            )
            rdma_b.start()

            rdma_a.wait_recv()
            fwd_y = pltpu.make_async_remote_copy(
                src_ref=out_ref.at[:, pl.ds(k * _R, _R), :],
                dst_ref=out_ref.at[:, pl.ds(k * _R, _R), :],
                send_sem=fy_send, recv_sem=fy_recv,
                device_id=y_nbr, device_id_type=pl.DeviceIdType.MESH,
            )
            fwd_y.start()
            fwd_z = pltpu.make_async_remote_copy(
                src_ref=out_ref.at[:, pl.ds(k * _R, _R), :],
                dst_ref=out_ref.at[:, pl.ds(k * _R, _R), :],
                send_sem=fz_send, recv_sem=fz_recv,
                device_id=z_nbr, device_id_type=pl.DeviceIdType.MESH,
            )
            fwd_z.start()

            rdma_b.wait_recv()
            fwd_y.wait_recv()
            fwd_z.wait_recv()
            rdma_a.wait_send()
            rdma_b.wait_send()
            fwd_y.wait_send()
            fwd_z.wait_send()

        @pl.when(jnp.logical_not(is_remote))
        def _():
            copy = pltpu.make_async_copy(x_ref, out_ref, copy_sem)
            copy.start()
            copy.wait()

    return pl.pallas_call(
        body,
        out_shape=jax.ShapeDtypeStruct(x.shape, x.dtype),
        in_specs=[
            pl.BlockSpec(memory_space=pltpu.SMEM),
            pl.BlockSpec(memory_space=pltpu.VMEM),
        ],
        out_specs=pl.BlockSpec(memory_space=pltpu.VMEM),
        scratch_shapes=[pltpu.SemaphoreType.DMA] * 9,
        compiler_params=pltpu.CompilerParams(collective_id=0),
    )(pi, x)


# device time: 14906 ns/iter; 1.2392x vs baseline; 1.0904x over previous
import jax
import jax.numpy as jnp
from jax import lax
from jax.experimental import pallas as pl
from jax.experimental.pallas import tpu as pltpu

_R = 128
_NCH = 2
_CR = _R // _NCH


def kernel(x, pi):
    def body(pi_ref, x_ref, out_ref,
             xa_send, xa_recv, xb_send, xb_recv,
             fy_send, fy_recv, fz_send, fz_recv, copy_sem):
        my_x = lax.axis_index("x")
        my_y = lax.axis_index("y")
        my_z = lax.axis_index("z")
        tgt = pi_ref[my_x]
        is_remote = tgt != my_x

        k = 2 * my_y + my_z
        kd = 3 - k

        partner = (tgt, my_y, my_z)
        y_nbr = (my_x, 1 - my_y, my_z)
        z_nbr = (my_x, my_y, 1 - my_z)

        @pl.when(is_remote)
        def _():
            barrier = pltpu.get_barrier_semaphore()
            for dev in (partner, y_nbr, z_nbr):
                pl.semaphore_signal(
                    barrier, inc=1,
                    device_id=dev, device_id_type=pl.DeviceIdType.MESH,
                )
            pl.semaphore_wait(barrier, 3)

            a_chunks = []
            for ch in range(_NCH):
                row = k * _R + ch * _CR
                rdma = pltpu.make_async_remote_copy(
                    src_ref=x_ref.at[:, pl.ds(row, _CR), :],
                    dst_ref=out_ref.at[:, pl.ds(row, _CR), :],
                    send_sem=xa_send.at[ch], recv_sem=xa_recv.at[ch],
                    device_id=partner, device_id_type=pl.DeviceIdType.MESH,
                )
                rdma.start()
                a_chunks.append(rdma)
            rdma_b = pltpu.make_async_remote_copy(
                src_ref=x_ref.at[:, pl.ds(kd * _R, _R), :],
                dst_ref=out_ref.at[:, pl.ds(kd * _R, _R), :],
                send_sem=xb_send, recv_sem=xb_recv,
                device_id=partner, device_id_type=pl.DeviceIdType.MESH,
            )
            rdma_b.start()

            forwards = []
            for ch in range(_NCH):
                row = k * _R + ch * _CR
                a_chunks[ch].wait_recv()
                for dev, ss, rs in (
                    (y_nbr, fy_send, fy_recv),
                    (z_nbr, fz_send, fz_recv),
                ):
                    fwd = pltpu.make_async_remote_copy(
                        src_ref=out_ref.at[:, pl.ds(row, _CR), :],
                        dst_ref=out_ref.at[:, pl.ds(row, _CR), :],
                        send_sem=ss.at[ch], recv_sem=rs.at[ch],
                        device_id=dev, device_id_type=pl.DeviceIdType.MESH,
                    )
                    fwd.start()
                    forwards.append(fwd)

            rdma_b.wait_recv()
            for fwd in forwards:
                fwd.wait_recv()
            for rdma in a_chunks:
                rdma.wait_send()
            rdma_b.wait_send()
            for fwd in forwards:
                fwd.wait_send()

        @pl.when(jnp.logical_not(is_remote))
        def _():
            copy = pltpu.make_async_copy(x_ref, out_ref, copy_sem)
            copy.start()
            copy.wait()

    return pl.pallas_call(
        body,
        out_shape=jax.ShapeDtypeStruct(x.shape, x.dtype),
        in_specs=[
            pl.BlockSpec(memory_space=pltpu.SMEM),
            pl.BlockSpec(memory_space=pltpu.VMEM),
        ],
        out_specs=pl.BlockSpec(memory_space=pltpu.VMEM),
        scratch_shapes=[
            pltpu.SemaphoreType.DMA((_NCH,)),
            pltpu.SemaphoreType.DMA((_NCH,)),
            pltpu.SemaphoreType.DMA,
            pltpu.SemaphoreType.DMA,
            pltpu.SemaphoreType.DMA((_NCH,)),
            pltpu.SemaphoreType.DMA((_NCH,)),
            pltpu.SemaphoreType.DMA((_NCH,)),
            pltpu.SemaphoreType.DMA((_NCH,)),
            pltpu.SemaphoreType.DMA,
        ],
        compiler_params=pltpu.CompilerParams(collective_id=0),
    )(pi, x)


# device time: 14362 ns/iter; 1.2862x vs baseline; 1.0379x over previous
import jax
import jax.numpy as jnp
from jax import lax
from jax.experimental import pallas as pl
from jax.experimental.pallas import tpu as pltpu

_R = 128
_NCH = 4
_CR = _R // _NCH


def kernel(x, pi):
    def body(pi_ref, x_ref, out_ref,
             xa_send, xa_recv, xb_send, xb_recv,
             fy_send, fy_recv, fz_send, fz_recv, copy_sem):
        my_x = lax.axis_index("x")
        my_y = lax.axis_index("y")
        my_z = lax.axis_index("z")
        tgt = pi_ref[my_x]
        is_remote = tgt != my_x

        k = 2 * my_y + my_z
        kd = 3 - k

        partner = (tgt, my_y, my_z)
        y_nbr = (my_x, 1 - my_y, my_z)
        z_nbr = (my_x, my_y, 1 - my_z)

        @pl.when(is_remote)
        def _():
            barrier = pltpu.get_barrier_semaphore()
            for dev in (partner, y_nbr, z_nbr):
                pl.semaphore_signal(
                    barrier, inc=1,
                    device_id=dev, device_id_type=pl.DeviceIdType.MESH,
                )
            pl.semaphore_wait(barrier, 3)

            a_chunks = []
            for ch in range(_NCH):
                row = k * _R + ch * _CR
                rdma = pltpu.make_async_remote_copy(
                    src_ref=x_ref.at[:, pl.ds(row, _CR), :],
                    dst_ref=out_ref.at[:, pl.ds(row, _CR), :],
                    send_sem=xa_send.at[ch], recv_sem=xa_recv.at[ch],
                    device_id=partner, device_id_type=pl.DeviceIdType.MESH,
                )
                rdma.start()
                a_chunks.append(rdma)
            rdma_b = pltpu.make_async_remote_copy(
                src_ref=x_ref.at[:, pl.ds(kd * _R, _R), :],
                dst_ref=out_ref.at[:, pl.ds(kd * _R, _R), :],
                send_sem=xb_send, recv_sem=xb_recv,
                device_id=partner, device_id_type=pl.DeviceIdType.MESH,
            )
            rdma_b.start()

            forwards = []
            for ch in range(_NCH):
                row = k * _R + ch * _CR
                a_chunks[ch].wait_recv()
                for dev, ss, rs in (
                    (y_nbr, fy_send, fy_recv),
                    (z_nbr, fz_send, fz_recv),
                ):
                    fwd = pltpu.make_async_remote_copy(
                        src_ref=out_ref.at[:, pl.ds(row, _CR), :],
                        dst_ref=out_ref.at[:, pl.ds(row, _CR), :],
                        send_sem=ss.at[ch], recv_sem=rs.at[ch],
                        device_id=dev, device_id_type=pl.DeviceIdType.MESH,
                    )
                    fwd.start()
                    forwards.append(fwd)

            rdma_b.wait_recv()
            for fwd in forwards:
                fwd.wait_recv()
            for rdma in a_chunks:
                rdma.wait_send()
            rdma_b.wait_send()
            for fwd in forwards:
                fwd.wait_send()

        @pl.when(jnp.logical_not(is_remote))
        def _():
            copy = pltpu.make_async_copy(x_ref, out_ref, copy_sem)
            copy.start()
            copy.wait()

    return pl.pallas_call(
        body,
        out_shape=jax.ShapeDtypeStruct(x.shape, x.dtype),
        in_specs=[
            pl.BlockSpec(memory_space=pltpu.SMEM),
            pl.BlockSpec(memory_space=pltpu.VMEM),
        ],
        out_specs=pl.BlockSpec(memory_space=pltpu.VMEM),
        scratch_shapes=[
            pltpu.SemaphoreType.DMA((_NCH,)),
            pltpu.SemaphoreType.DMA((_NCH,)),
            pltpu.SemaphoreType.DMA,
            pltpu.SemaphoreType.DMA,
            pltpu.SemaphoreType.DMA((_NCH,)),
            pltpu.SemaphoreType.DMA((_NCH,)),
            pltpu.SemaphoreType.DMA((_NCH,)),
            pltpu.SemaphoreType.DMA((_NCH,)),
            pltpu.SemaphoreType.DMA,
        ],
        compiler_params=pltpu.CompilerParams(collective_id=0),
    )(pi, x)
